# Initial kernel scaffold; baseline (speedup 1.0000x reference)
import jax
import jax.numpy as jnp
from jax.experimental import pallas as pl


def kernel(features, edge_index, W0, b0, W1, b1, Wp, bp):
    raise NotImplementedError("write your pallas kernel here")


# speedup vs baseline: 7.4481x; 7.4481x over previous
import functools

import jax
import jax.numpy as jnp
from jax import lax
from jax.experimental import pallas as pl
from jax.experimental.pallas import tpu as pltpu
from jax.experimental.pallas import tpu_sc as plsc

N = 10000
E = 320000
D = 128
NC = 2
NS = 16
NW = NC * NS
CH = 80
EW = CH * 128
E_PAD = NW * EW
N_PAD = 10240
ROWS_Z = N_PAD // NS
ROWS_O = 632
ROWS_LAST = N - 15 * ROWS_O
BN = 2000
GRID = N // BN


def _mesh():
    return plsc.VectorSubcoreMesh(core_axis_name="c", subcore_axis_name="s",
                                  num_cores=NC, num_subcores=NS)


def _sc_spmm_body(g_hbm, src_hbm, dst_hbm, zero_hbm, out_hbm,
                  src_v, dst_v, rows_v, acc_sh, sem):
    c = lax.axis_index("c")
    s = lax.axis_index("s")
    w = c * NS + s
    zo = pl.multiple_of(s * ROWS_Z, 128)
    pltpu.sync_copy(src_hbm.at[w], src_v)
    pltpu.sync_copy(dst_hbm.at[w], dst_v)
    pltpu.sync_copy(zero_hbm.at[pl.ds(zo, ROWS_Z)],
                    acc_sh.at[pl.ds(zo, ROWS_Z)])
    plsc.subcore_barrier()

    def body(j, carry):
        pltpu.async_copy(g_hbm.at[src_v.at[j]], rows_v, sem).wait()
        pltpu.sync_copy(rows_v, acc_sh.at[dst_v.at[j]], add=True)
        return carry

    lax.fori_loop(0, CH, body, 0)
    plsc.subcore_barrier()
    oo = pl.multiple_of(s * ROWS_O, 8)
    @pl.when(s < 15)
    def _():
        pltpu.sync_copy(acc_sh.at[pl.ds(oo, ROWS_O)],
                        out_hbm.at[c].at[pl.ds(oo, ROWS_O)])
    @pl.when(s == 15)
    def _():
        pltpu.sync_copy(acc_sh.at[pl.ds(15 * ROWS_O, ROWS_LAST)],
                        out_hbm.at[c].at[pl.ds(15 * ROWS_O, ROWS_LAST)])


@functools.cache
def _sc_spmm():
    return pl.kernel(
        _sc_spmm_body,
        out_type=jax.ShapeDtypeStruct((NC, N, D), jnp.float32),
        mesh=_mesh(),
        scratch_types=[
            pltpu.VMEM((CH, 128), jnp.int32),
            pltpu.VMEM((CH, 128), jnp.int32),
            pltpu.VMEM((128, D), jnp.float32),
            pltpu.VMEM_SHARED((N_PAD, D), jnp.float32),
            pltpu.SemaphoreType.DMA,
        ],
    )


def _sc_degree_body(dst_hbm, zero_hbm, out_hbm, dst_v, ones_v, acc_sh, sem):
    c = lax.axis_index("c")
    s = lax.axis_index("s")
    w = c * NS + s
    zo = pl.multiple_of(s * ROWS_Z, 128)
    pltpu.sync_copy(dst_hbm.at[w], dst_v)
    for i in range(8):
        ones_v[pl.ds(i * 16, 16)] = jnp.ones((16,), jnp.float32)
    pltpu.sync_copy(zero_hbm.at[pl.ds(zo, ROWS_Z)],
                    acc_sh.at[pl.ds(zo, ROWS_Z)])
    plsc.subcore_barrier()

    def body(j, carry):
        pltpu.sync_copy(ones_v, acc_sh.at[dst_v.at[j]], add=True)
        return carry

    lax.fori_loop(0, CH, body, 0)
    plsc.subcore_barrier()
    fo = pl.multiple_of(c * N_PAD + s * ROWS_Z, 128)
    pltpu.sync_copy(acc_sh.at[pl.ds(zo, ROWS_Z)],
                    out_hbm.at[pl.ds(fo, ROWS_Z)])


@functools.cache
def _sc_degree():
    return pl.kernel(
        _sc_degree_body,
        out_type=jax.ShapeDtypeStruct((NC * N_PAD,), jnp.float32),
        mesh=_mesh(),
        scratch_types=[
            pltpu.VMEM((CH, 128), jnp.int32),
            pltpu.VMEM((128,), jnp.float32),
            pltpu.VMEM_SHARED((N_PAD,), jnp.float32),
            pltpu.SemaphoreType.DMA,
        ],
    )


def _dcol(deg_ref):
    dsum = deg_ref[0] + deg_ref[1]
    return jax.lax.rsqrt(jnp.maximum(dsum, 1.0))


def _leaky(x):
    return jnp.where(x >= 0.0, x, 0.01 * x)


_deg_spec = pl.BlockSpec((2, BN, 1), lambda i: (0, i, 0))
_row_spec = pl.BlockSpec((BN, D), lambda i: (i, 0))
_par_spec = pl.BlockSpec((2, BN, D), lambda i: (0, i, 0))
_full = lambda shape: pl.BlockSpec(shape, lambda i: tuple(0 for _ in shape))


def _tc_scale_body(deg_ref, x_ref, o_ref):
    o_ref[...] = x_ref[...] * _dcol(deg_ref)


def _tc_mid_body(deg_ref, p_ref, o_ref):
    d = _dcol(deg_ref)
    o_ref[...] = (p_ref[0] + p_ref[1]) * (d * d)


def _tc_combine_body(deg_ref, h_ref, p1_ref, p2_ref, w_ref, b_ref,
                     h1_ref, g2_ref):
    d = _dcol(deg_ref)
    h = h_ref[...]
    u1 = (p1_ref[0] + p1_ref[1]) * d
    u2 = (p2_ref[0] + p2_ref[1]) * d
    acc = jnp.dot(h, w_ref[0] - w_ref[2], preferred_element_type=jnp.float32)
    acc = acc - jnp.dot(u1, w_ref[1], preferred_element_type=jnp.float32)
    acc = acc + 2.0 * jnp.dot(u2, w_ref[2], preferred_element_type=jnp.float32)
    acc = acc + b_ref[...]
    h1 = _leaky(acc)
    h1_ref[...] = h1
    g2_ref[...] = h1 * d


def _tc_final_body(deg_ref, h_ref, p1_ref, p2_ref, w_ref, b_ref,
                   wp_ref, bp_ref, o_ref):
    d = _dcol(deg_ref)
    h = h_ref[...]
    u1 = (p1_ref[0] + p1_ref[1]) * d
    u2 = (p2_ref[0] + p2_ref[1]) * d
    acc = jnp.dot(h, w_ref[0] - w_ref[2], preferred_element_type=jnp.float32)
    acc = acc - jnp.dot(u1, w_ref[1], preferred_element_type=jnp.float32)
    acc = acc + 2.0 * jnp.dot(u2, w_ref[2], preferred_element_type=jnp.float32)
    h2 = _leaky(acc + b_ref[...])
    o_ref[...] = (jnp.dot(h2, wp_ref[...], preferred_element_type=jnp.float32)
                  + bp_ref[...])


def _tc_scale(degc, x):
    return pl.pallas_call(
        _tc_scale_body,
        grid=(GRID,),
        in_specs=[_deg_spec, _row_spec],
        out_specs=_row_spec,
        out_shape=jax.ShapeDtypeStruct((N, D), jnp.float32),
    )(degc, x)


def _tc_mid(degc, p):
    return pl.pallas_call(
        _tc_mid_body,
        grid=(GRID,),
        in_specs=[_deg_spec, _par_spec],
        out_specs=_row_spec,
        out_shape=jax.ShapeDtypeStruct((N, D), jnp.float32),
    )(degc, p)


def _tc_combine(degc, h, p1, p2, W, b):
    return pl.pallas_call(
        _tc_combine_body,
        grid=(GRID,),
        in_specs=[_deg_spec, _row_spec, _par_spec, _par_spec,
                  _full((3, D, D)), _full((1, D))],
        out_specs=[_row_spec, _row_spec],
        out_shape=[jax.ShapeDtypeStruct((N, D), jnp.float32),
                   jax.ShapeDtypeStruct((N, D), jnp.float32)],
    )(degc, h, p1, p2, W, b)


def _tc_final(degc, h, p1, p2, W, b, Wp, bp):
    return pl.pallas_call(
        _tc_final_body,
        grid=(GRID,),
        in_specs=[_deg_spec, _row_spec, _par_spec, _par_spec,
                  _full((3, D, D)), _full((1, D)), _full((D, 1)),
                  _full((1, 1))],
        out_specs=pl.BlockSpec((BN, 1), lambda i: (i, 0)),
        out_shape=jax.ShapeDtypeStruct((N, 1), jnp.float32),
    )(degc, h, p1, p2, W, b, Wp, bp)


def kernel(features, edge_index, W0, b0, W1, b1, Wp, bp):
    src = edge_index[0]
    dst = edge_index[1]
    pad = E_PAD - E
    pad_src = (jnp.arange(pad, dtype=jnp.int32) * 131) % N
    pad_dst = N + (jnp.arange(pad, dtype=jnp.int32) % 64)
    src3 = jnp.concatenate([src, pad_src]).reshape(NW, CH, 128)
    dst3 = jnp.concatenate([dst, pad_dst]).reshape(NW, CH, 128)
    zeros2d = jnp.zeros((N_PAD, D), jnp.float32)
    zeros1d = jnp.zeros((N_PAD,), jnp.float32)

    degp = _sc_degree()(dst3, zeros1d).reshape(NC, N_PAD)
    degc = degp[:, :N, None]

    spmm = _sc_spmm()
    g0 = _tc_scale(degc, features)
    p1 = spmm(g0, src3, dst3, zeros2d)
    g1 = _tc_mid(degc, p1)
    p2 = spmm(g1, src3, dst3, zeros2d)
    h1, g2 = _tc_combine(degc, features, p1, p2, W0, b0.reshape(1, D))

    q1 = spmm(g2, src3, dst3, zeros2d)
    g3 = _tc_mid(degc, q1)
    q2 = spmm(g3, src3, dst3, zeros2d)
    return _tc_final(degc, h1, q1, q2, W1, b1.reshape(1, D),
                     Wp, bp.reshape(1, 1))


# speedup vs baseline: 7.4513x; 1.0004x over previous
import functools

import jax
import jax.numpy as jnp
from jax import lax
from jax.experimental import pallas as pl
from jax.experimental.pallas import tpu as pltpu
from jax.experimental.pallas import tpu_sc as plsc

N = 10000
E = 320000
D = 128
NC = 2
NS = 16
NW = NC * NS
CK = 128
CH = 80
EW = CH * CK
E_PAD = NW * EW
N_PAD = 10240
ROWS_Z = N_PAD // NS
ROWS_O = 632
ROWS_LAST = N - 15 * ROWS_O
BN = 2000
GRID = N // BN


def _mesh():
    return plsc.VectorSubcoreMesh(core_axis_name="c", subcore_axis_name="s",
                                  num_cores=NC, num_subcores=NS)


def _sc_spmm_body(g_hbm, src_hbm, dst_hbm, zero_hbm, out_hbm,
                  src_v, dst_v, b0, acc_sh, semga):
    c = lax.axis_index("c")
    s = lax.axis_index("s")
    w = c * NS + s
    zo = pl.multiple_of(s * ROWS_Z, 128)
    pltpu.sync_copy(src_hbm.at[w], src_v)
    pltpu.sync_copy(dst_hbm.at[w], dst_v)
    pltpu.sync_copy(zero_hbm.at[pl.ds(zo, ROWS_Z)],
                    acc_sh.at[pl.ds(zo, ROWS_Z)])
    plsc.subcore_barrier()

    def body(j, carry):
        pltpu.async_copy(g_hbm.at[src_v.at[j]], b0, semga).wait()
        pltpu.sync_copy(b0, acc_sh.at[dst_v.at[j]], add=True)
        return carry

    lax.fori_loop(0, CH, body, 0)
    plsc.subcore_barrier()
    oo = pl.multiple_of(s * ROWS_O, 8)
    @pl.when(s < 15)
    def _():
        pltpu.sync_copy(acc_sh.at[pl.ds(oo, ROWS_O)],
                        out_hbm.at[c].at[pl.ds(oo, ROWS_O)])
    @pl.when(s == 15)
    def _():
        pltpu.sync_copy(acc_sh.at[pl.ds(15 * ROWS_O, ROWS_LAST)],
                        out_hbm.at[c].at[pl.ds(15 * ROWS_O, ROWS_LAST)])


@functools.cache
def _sc_spmm():
    return pl.kernel(
        _sc_spmm_body,
        out_type=jax.ShapeDtypeStruct((NC, N, D), jnp.float32),
        mesh=_mesh(),
        scratch_types=[
            pltpu.VMEM((CH, CK), jnp.int32),
            pltpu.VMEM((CH, CK), jnp.int32),
            pltpu.VMEM((CK, D), jnp.float32),
            pltpu.VMEM_SHARED((N_PAD, D), jnp.float32),
            pltpu.SemaphoreType.DMA,
        ],
    )


def _sc_degree_body(dst_hbm, zero_hbm, out_hbm, dst_v, ones_v, acc_sh, sem):
    c = lax.axis_index("c")
    s = lax.axis_index("s")
    w = c * NS + s
    zo = pl.multiple_of(s * ROWS_Z, 128)
    pltpu.sync_copy(dst_hbm.at[w], dst_v)
    for i in range(CK // 16):
        ones_v[pl.ds(i * 16, 16)] = jnp.ones((16,), jnp.float32)
    pltpu.sync_copy(zero_hbm.at[pl.ds(zo, ROWS_Z)],
                    acc_sh.at[pl.ds(zo, ROWS_Z)])
    plsc.subcore_barrier()

    def body(j, carry):
        pltpu.sync_copy(ones_v, acc_sh.at[dst_v.at[j]], add=True)
        return carry

    lax.fori_loop(0, CH, body, 0)
    plsc.subcore_barrier()
    fo = pl.multiple_of(c * N_PAD + s * ROWS_Z, 128)
    pltpu.sync_copy(acc_sh.at[pl.ds(zo, ROWS_Z)],
                    out_hbm.at[pl.ds(fo, ROWS_Z)])


@functools.cache
def _sc_degree():
    return pl.kernel(
        _sc_degree_body,
        out_type=jax.ShapeDtypeStruct((NC * N_PAD,), jnp.float32),
        mesh=_mesh(),
        scratch_types=[
            pltpu.VMEM((CH, CK), jnp.int32),
            pltpu.VMEM((CK,), jnp.float32),
            pltpu.VMEM_SHARED((N_PAD,), jnp.float32),
            pltpu.SemaphoreType.DMA,
        ],
    )


def _dcol(deg_ref):
    dsum = deg_ref[0] + deg_ref[1]
    return jax.lax.rsqrt(jnp.maximum(dsum, 1.0))


def _leaky(x):
    return jnp.where(x >= 0.0, x, 0.01 * x)


_deg_spec = pl.BlockSpec((2, BN, 1), lambda i: (0, i, 0))
_row_spec = pl.BlockSpec((BN, D), lambda i: (i, 0))
_par_spec = pl.BlockSpec((2, BN, D), lambda i: (0, i, 0))
_full = lambda shape: pl.BlockSpec(shape, lambda i: tuple(0 for _ in shape))


def _tc_scale_body(deg_ref, x_ref, o_ref):
    o_ref[...] = x_ref[...] * _dcol(deg_ref)


def _tc_mid_body(deg_ref, p_ref, o_ref):
    d = _dcol(deg_ref)
    o_ref[...] = (p_ref[0] + p_ref[1]) * (d * d)


def _tc_combine_body(deg_ref, h_ref, p1_ref, p2_ref, w_ref, b_ref,
                     h1_ref, g2_ref):
    d = _dcol(deg_ref)
    h = h_ref[...]
    u1 = (p1_ref[0] + p1_ref[1]) * d
    u2 = (p2_ref[0] + p2_ref[1]) * d
    acc = jnp.dot(h, w_ref[0] - w_ref[2], preferred_element_type=jnp.float32)
    acc = acc - jnp.dot(u1, w_ref[1], preferred_element_type=jnp.float32)
    acc = acc + 2.0 * jnp.dot(u2, w_ref[2], preferred_element_type=jnp.float32)
    acc = acc + b_ref[...]
    h1 = _leaky(acc)
    h1_ref[...] = h1
    g2_ref[...] = h1 * d


def _tc_final_body(deg_ref, h_ref, p1_ref, p2_ref, w_ref, b_ref,
                   wp_ref, bp_ref, o_ref):
    d = _dcol(deg_ref)
    h = h_ref[...]
    u1 = (p1_ref[0] + p1_ref[1]) * d
    u2 = (p2_ref[0] + p2_ref[1]) * d
    acc = jnp.dot(h, w_ref[0] - w_ref[2], preferred_element_type=jnp.float32)
    acc = acc - jnp.dot(u1, w_ref[1], preferred_element_type=jnp.float32)
    acc = acc + 2.0 * jnp.dot(u2, w_ref[2], preferred_element_type=jnp.float32)
    h2 = _leaky(acc + b_ref[...])
    o_ref[...] = (jnp.dot(h2, wp_ref[...], preferred_element_type=jnp.float32)
                  + bp_ref[...])


def _tc_scale(degc, x):
    return pl.pallas_call(
        _tc_scale_body,
        grid=(GRID,),
        in_specs=[_deg_spec, _row_spec],
        out_specs=_row_spec,
        out_shape=jax.ShapeDtypeStruct((N, D), jnp.float32),
    )(degc, x)


def _tc_mid(degc, p):
    return pl.pallas_call(
        _tc_mid_body,
        grid=(GRID,),
        in_specs=[_deg_spec, _par_spec],
        out_specs=_row_spec,
        out_shape=jax.ShapeDtypeStruct((N, D), jnp.float32),
    )(degc, p)


def _tc_combine(degc, h, p1, p2, W, b):
    return pl.pallas_call(
        _tc_combine_body,
        grid=(GRID,),
        in_specs=[_deg_spec, _row_spec, _par_spec, _par_spec,
                  _full((3, D, D)), _full((1, D))],
        out_specs=[_row_spec, _row_spec],
        out_shape=[jax.ShapeDtypeStruct((N, D), jnp.float32),
                   jax.ShapeDtypeStruct((N, D), jnp.float32)],
    )(degc, h, p1, p2, W, b)


def _tc_final(degc, h, p1, p2, W, b, Wp, bp):
    return pl.pallas_call(
        _tc_final_body,
        grid=(GRID,),
        in_specs=[_deg_spec, _row_spec, _par_spec, _par_spec,
                  _full((3, D, D)), _full((1, D)), _full((D, 1)),
                  _full((1, 1))],
        out_specs=pl.BlockSpec((BN, 1), lambda i: (i, 0)),
        out_shape=jax.ShapeDtypeStruct((N, 1), jnp.float32),
    )(degc, h, p1, p2, W, b, Wp, bp)


def kernel(features, edge_index, W0, b0, W1, b1, Wp, bp):
    src = edge_index[0]
    dst = edge_index[1]
    pad = E_PAD - E
    pad_src = (jnp.arange(pad, dtype=jnp.int32) * 131) % N
    pad_dst = N + (jnp.arange(pad, dtype=jnp.int32) % 64)
    src3 = jnp.concatenate([src, pad_src]).reshape(NW, CH, CK)
    dst3 = jnp.concatenate([dst, pad_dst]).reshape(NW, CH, CK)
    zeros2d = jnp.zeros((N_PAD, D), jnp.float32)
    zeros1d = jnp.zeros((N_PAD,), jnp.float32)

    degp = _sc_degree()(dst3, zeros1d).reshape(NC, N_PAD)
    degc = degp[:, :N, None]

    spmm = _sc_spmm()
    g0 = _tc_scale(degc, features)
    p1 = spmm(g0, src3, dst3, zeros2d)
    g1 = _tc_mid(degc, p1)
    p2 = spmm(g1, src3, dst3, zeros2d)
    h1, g2 = _tc_combine(degc, features, p1, p2, W0, b0.reshape(1, D))

    q1 = spmm(g2, src3, dst3, zeros2d)
    g3 = _tc_mid(degc, q1)
    q2 = spmm(g3, src3, dst3, zeros2d)
    return _tc_final(degc, h1, q1, q2, W1, b1.reshape(1, D),
                     Wp, bp.reshape(1, 1))


# speedup vs baseline: 7.5599x; 1.0146x over previous
import functools

import jax
import jax.numpy as jnp
from jax import lax
from jax.experimental import pallas as pl
from jax.experimental.pallas import tpu as pltpu
from jax.experimental.pallas import tpu_sc as plsc

N = 10000
E = 320000
D = 128
NC = 2
NS = 16
NW = NC * NS
CK = 128
CH = 80
EW = CH * CK
E_PAD = NW * EW
N_PAD = 10240
ROWS_Z = N_PAD // NS
ROWS_O = 632
ROWS_LAST = N - 15 * ROWS_O
BN = 2000
GRID = N // BN


def _mesh():
    return plsc.VectorSubcoreMesh(core_axis_name="c", subcore_axis_name="s",
                                  num_cores=NC, num_subcores=NS)


def _sc_spmm_body(g_hbm, src_hbm, dst_hbm, out_hbm,
                  src_v, dst_v, b0, acc_sh, semga):
    c = lax.axis_index("c")
    s = lax.axis_index("s")
    w = c * NS + s
    zo = pl.multiple_of(s * ROWS_Z, 128)
    pltpu.sync_copy(src_hbm.at[w], src_v)
    pltpu.sync_copy(dst_hbm.at[w], dst_v)

    def zrow(j, carry):
        for k in range(D // 16):
            b0[j, pl.ds(k * 16, 16)] = jnp.zeros((16,), jnp.float32)
        return carry

    lax.fori_loop(0, CK, zrow, 0)
    for j in range(ROWS_Z // CK):
        pltpu.sync_copy(b0, acc_sh.at[pl.ds(zo + j * CK, CK)])
    plsc.subcore_barrier()

    def body(j, carry):
        pltpu.async_copy(g_hbm.at[src_v.at[j]], b0, semga).wait()
        pltpu.sync_copy(b0, acc_sh.at[dst_v.at[j]], add=True)
        return carry

    lax.fori_loop(0, CH, body, 0)
    plsc.subcore_barrier()
    oo = pl.multiple_of(s * ROWS_O, 8)
    @pl.when(s < 15)
    def _():
        pltpu.sync_copy(acc_sh.at[pl.ds(oo, ROWS_O)],
                        out_hbm.at[c].at[pl.ds(oo, ROWS_O)])
    @pl.when(s == 15)
    def _():
        pltpu.sync_copy(acc_sh.at[pl.ds(15 * ROWS_O, ROWS_LAST)],
                        out_hbm.at[c].at[pl.ds(15 * ROWS_O, ROWS_LAST)])


@functools.cache
def _sc_spmm():
    return pl.kernel(
        _sc_spmm_body,
        out_type=jax.ShapeDtypeStruct((NC, N, D), jnp.float32),
        mesh=_mesh(),
        scratch_types=[
            pltpu.VMEM((CH, CK), jnp.int32),
            pltpu.VMEM((CH, CK), jnp.int32),
            pltpu.VMEM((CK, D), jnp.float32),
            pltpu.VMEM_SHARED((N_PAD, D), jnp.float32),
            pltpu.SemaphoreType.DMA,
        ],
    )


def _sc_degree_body(dst_hbm, zero_hbm, out_hbm, dst_v, ones_v, acc_sh, sem):
    c = lax.axis_index("c")
    s = lax.axis_index("s")
    w = c * NS + s
    zo = pl.multiple_of(s * ROWS_Z, 128)
    pltpu.sync_copy(dst_hbm.at[w], dst_v)
    for i in range(CK // 16):
        ones_v[pl.ds(i * 16, 16)] = jnp.ones((16,), jnp.float32)
    pltpu.sync_copy(zero_hbm.at[pl.ds(zo, ROWS_Z)],
                    acc_sh.at[pl.ds(zo, ROWS_Z)])
    plsc.subcore_barrier()

    def body(j, carry):
        pltpu.sync_copy(ones_v, acc_sh.at[dst_v.at[j]], add=True)
        return carry

    lax.fori_loop(0, CH, body, 0)
    plsc.subcore_barrier()
    fo = pl.multiple_of(c * N_PAD + s * ROWS_Z, 128)
    pltpu.sync_copy(acc_sh.at[pl.ds(zo, ROWS_Z)],
                    out_hbm.at[pl.ds(fo, ROWS_Z)])


@functools.cache
def _sc_degree():
    return pl.kernel(
        _sc_degree_body,
        out_type=jax.ShapeDtypeStruct((NC * N_PAD,), jnp.float32),
        mesh=_mesh(),
        scratch_types=[
            pltpu.VMEM((CH, CK), jnp.int32),
            pltpu.VMEM((CK,), jnp.float32),
            pltpu.VMEM_SHARED((N_PAD,), jnp.float32),
            pltpu.SemaphoreType.DMA,
        ],
    )


def _dcol(deg_ref):
    dsum = deg_ref[0] + deg_ref[1]
    return jax.lax.rsqrt(jnp.maximum(dsum, 1.0))


def _leaky(x):
    return jnp.where(x >= 0.0, x, 0.01 * x)


_deg_spec = pl.BlockSpec((2, BN, 1), lambda i: (0, i, 0))
_row_spec = pl.BlockSpec((BN, D), lambda i: (i, 0))
_par_spec = pl.BlockSpec((2, BN, D), lambda i: (0, i, 0))
_full = lambda shape: pl.BlockSpec(shape, lambda i: tuple(0 for _ in shape))


def _tc_scale_body(deg_ref, x_ref, o_ref):
    o_ref[...] = x_ref[...] * _dcol(deg_ref)


def _tc_mid_body(deg_ref, p_ref, o_ref):
    d = _dcol(deg_ref)
    o_ref[...] = (p_ref[0] + p_ref[1]) * (d * d)


def _tc_combine_body(deg_ref, h_ref, p1_ref, p2_ref, w_ref, b_ref,
                     h1_ref, g2_ref):
    d = _dcol(deg_ref)
    h = h_ref[...]
    u1 = (p1_ref[0] + p1_ref[1]) * d
    u2 = (p2_ref[0] + p2_ref[1]) * d
    acc = jnp.dot(h, w_ref[0] - w_ref[2], preferred_element_type=jnp.float32)
    acc = acc - jnp.dot(u1, w_ref[1], preferred_element_type=jnp.float32)
    acc = acc + 2.0 * jnp.dot(u2, w_ref[2], preferred_element_type=jnp.float32)
    acc = acc + b_ref[...]
    h1 = _leaky(acc)
    h1_ref[...] = h1
    g2_ref[...] = h1 * d


def _tc_final_body(deg_ref, h_ref, p1_ref, p2_ref, w_ref, b_ref,
                   wp_ref, bp_ref, o_ref):
    d = _dcol(deg_ref)
    h = h_ref[...]
    u1 = (p1_ref[0] + p1_ref[1]) * d
    u2 = (p2_ref[0] + p2_ref[1]) * d
    acc = jnp.dot(h, w_ref[0] - w_ref[2], preferred_element_type=jnp.float32)
    acc = acc - jnp.dot(u1, w_ref[1], preferred_element_type=jnp.float32)
    acc = acc + 2.0 * jnp.dot(u2, w_ref[2], preferred_element_type=jnp.float32)
    h2 = _leaky(acc + b_ref[...])
    o_ref[...] = (jnp.dot(h2, wp_ref[...], preferred_element_type=jnp.float32)
                  + bp_ref[...])


def _tc_scale(degc, x):
    return pl.pallas_call(
        _tc_scale_body,
        grid=(GRID,),
        in_specs=[_deg_spec, _row_spec],
        out_specs=_row_spec,
        out_shape=jax.ShapeDtypeStruct((N, D), jnp.float32),
    )(degc, x)


def _tc_mid(degc, p):
    return pl.pallas_call(
        _tc_mid_body,
        grid=(GRID,),
        in_specs=[_deg_spec, _par_spec],
        out_specs=_row_spec,
        out_shape=jax.ShapeDtypeStruct((N, D), jnp.float32),
    )(degc, p)


def _tc_combine(degc, h, p1, p2, W, b):
    return pl.pallas_call(
        _tc_combine_body,
        grid=(GRID,),
        in_specs=[_deg_spec, _row_spec, _par_spec, _par_spec,
                  _full((3, D, D)), _full((1, D))],
        out_specs=[_row_spec, _row_spec],
        out_shape=[jax.ShapeDtypeStruct((N, D), jnp.float32),
                   jax.ShapeDtypeStruct((N, D), jnp.float32)],
    )(degc, h, p1, p2, W, b)


def _tc_final(degc, h, p1, p2, W, b, Wp, bp):
    return pl.pallas_call(
        _tc_final_body,
        grid=(GRID,),
        in_specs=[_deg_spec, _row_spec, _par_spec, _par_spec,
                  _full((3, D, D)), _full((1, D)), _full((D, 1)),
                  _full((1, 1))],
        out_specs=pl.BlockSpec((BN, 1), lambda i: (i, 0)),
        out_shape=jax.ShapeDtypeStruct((N, 1), jnp.float32),
    )(degc, h, p1, p2, W, b, Wp, bp)


def kernel(features, edge_index, W0, b0, W1, b1, Wp, bp):
    src = edge_index[0]
    dst = edge_index[1]
    pad = E_PAD - E
    pad_src = (jnp.arange(pad, dtype=jnp.int32) * 131) % N
    pad_dst = N + (jnp.arange(pad, dtype=jnp.int32) % 64)
    src3 = jnp.concatenate([src, pad_src]).reshape(NW, CH, CK)
    dst3 = jnp.concatenate([dst, pad_dst]).reshape(NW, CH, CK)
    zeros1d = jnp.zeros((N_PAD,), jnp.float32)

    degp = _sc_degree()(dst3, zeros1d).reshape(NC, N_PAD)
    degc = degp[:, :N, None]

    spmm = _sc_spmm()
    g0 = _tc_scale(degc, features)
    p1 = spmm(g0, src3, dst3)
    g1 = _tc_mid(degc, p1)
    p2 = spmm(g1, src3, dst3)
    h1, g2 = _tc_combine(degc, features, p1, p2, W0, b0.reshape(1, D))

    q1 = spmm(g2, src3, dst3)
    g3 = _tc_mid(degc, q1)
    q2 = spmm(g3, src3, dst3)
    return _tc_final(degc, h1, q1, q2, W1, b1.reshape(1, D),
                     Wp, bp.reshape(1, 1))


# speedup vs baseline: 7.5691x; 1.0012x over previous
import functools

import jax
import jax.numpy as jnp
from jax import lax
from jax.experimental import pallas as pl
from jax.experimental.pallas import tpu as pltpu
from jax.experimental.pallas import tpu_sc as plsc

N = 10000
E = 320000
D = 128
NC = 2
NS = 16
NW = NC * NS
CK = 128
CH = 80
EW = CH * CK
E_PAD = NW * EW
N_PAD = 10240
ROWS_Z = N_PAD // NS
ROWS_O = 632
ROWS_LAST = N - 15 * ROWS_O
BN = 2000
GRID = N // BN


def _mesh():
    return plsc.VectorSubcoreMesh(core_axis_name="c", subcore_axis_name="s",
                                  num_cores=NC, num_subcores=NS)


def _sc_spmm_body(g_hbm, src_hbm, dst_hbm, out_hbm,
                  src_v, dst_v, b0, acc_sh, semga):
    c = lax.axis_index("c")
    s = lax.axis_index("s")
    w = c * NS + s
    zo = pl.multiple_of(s * ROWS_Z, 128)
    pltpu.sync_copy(src_hbm.at[w], src_v)
    pltpu.sync_copy(dst_hbm.at[w], dst_v)

    def zrow(j, carry):
        for k in range(D // 16):
            b0[j, pl.ds(k * 16, 16)] = jnp.zeros((16,), jnp.float32)
        return carry

    lax.fori_loop(0, CK, zrow, 0)
    for j in range(ROWS_Z // CK):
        pltpu.sync_copy(b0, acc_sh.at[pl.ds(zo + j * CK, CK)])
    plsc.subcore_barrier()

    def body(j, carry):
        pltpu.async_copy(g_hbm.at[src_v.at[j]], b0, semga).wait()
        pltpu.sync_copy(b0, acc_sh.at[dst_v.at[j]], add=True)
        return carry

    lax.fori_loop(0, CH, body, 0)
    plsc.subcore_barrier()
    oo = pl.multiple_of(s * ROWS_O, 8)
    @pl.when(s < 15)
    def _():
        pltpu.sync_copy(acc_sh.at[pl.ds(oo, ROWS_O)],
                        out_hbm.at[c].at[pl.ds(oo, ROWS_O)])
    @pl.when(s == 15)
    def _():
        pltpu.sync_copy(acc_sh.at[pl.ds(15 * ROWS_O, ROWS_LAST)],
                        out_hbm.at[c].at[pl.ds(15 * ROWS_O, ROWS_LAST)])


@functools.cache
def _sc_spmm():
    return pl.kernel(
        _sc_spmm_body,
        out_type=jax.ShapeDtypeStruct((NC, N, D), jnp.float32),
        mesh=_mesh(),
        scratch_types=[
            pltpu.VMEM((CH, CK), jnp.int32),
            pltpu.VMEM((CH, CK), jnp.int32),
            pltpu.VMEM((CK, D), jnp.float32),
            pltpu.VMEM_SHARED((N_PAD, D), jnp.float32),
            pltpu.SemaphoreType.DMA,
        ],
    )


def _sc_degree_body(dst_hbm, zero_hbm, out_hbm, dst_v, ones_v, acc_sh, sem):
    c = lax.axis_index("c")
    s = lax.axis_index("s")
    w = c * NS + s
    zo = pl.multiple_of(s * ROWS_Z, 128)
    pltpu.sync_copy(dst_hbm.at[w], dst_v)
    for i in range(CK // 16):
        ones_v[pl.ds(i * 16, 16)] = jnp.ones((16,), jnp.float32)
    pltpu.sync_copy(zero_hbm.at[pl.ds(zo, ROWS_Z)],
                    acc_sh.at[pl.ds(zo, ROWS_Z)])
    plsc.subcore_barrier()

    def body(j, carry):
        pltpu.sync_copy(ones_v, acc_sh.at[dst_v.at[j]], add=True)
        return carry

    lax.fori_loop(0, CH, body, 0)
    plsc.subcore_barrier()
    fo = pl.multiple_of(c * N_PAD + s * ROWS_Z, 128)
    pltpu.sync_copy(acc_sh.at[pl.ds(zo, ROWS_Z)],
                    out_hbm.at[pl.ds(fo, ROWS_Z)])


@functools.cache
def _sc_degree():
    return pl.kernel(
        _sc_degree_body,
        out_type=jax.ShapeDtypeStruct((NC * N_PAD,), jnp.float32),
        mesh=_mesh(),
        scratch_types=[
            pltpu.VMEM((CH, CK), jnp.int32),
            pltpu.VMEM((CK,), jnp.float32),
            pltpu.VMEM_SHARED((N_PAD,), jnp.float32),
            pltpu.SemaphoreType.DMA,
        ],
    )


def _dcol(deg_ref):
    dsum = deg_ref[0] + deg_ref[1]
    return jax.lax.rsqrt(jnp.maximum(dsum, 1.0))


def _leaky(x):
    return jnp.where(x >= 0.0, x, 0.01 * x)


_deg_spec = pl.BlockSpec((2, BN, 1), lambda i: (0, i, 0))
_row_spec = pl.BlockSpec((BN, D), lambda i: (i, 0))
_par_spec = pl.BlockSpec((2, BN, D), lambda i: (0, i, 0))
_full = lambda shape: pl.BlockSpec(shape, lambda i: tuple(0 for _ in shape))


def _tc_scale_body(deg_ref, x_ref, o_ref):
    o_ref[...] = x_ref[...] * _dcol(deg_ref)


def _tc_mid_body(deg_ref, p_ref, o_ref):
    d = _dcol(deg_ref)
    o_ref[...] = ((p_ref[0] + p_ref[1]) * d) * d


def _tc_combine_body(deg_ref, h_ref, p1_ref, p2_ref, w_ref, b_ref,
                     h1_ref, g2_ref):
    d = _dcol(deg_ref)
    h = h_ref[...]
    t1 = -((p1_ref[0] + p1_ref[1]) * d)
    t2 = 2.0 * ((p2_ref[0] + p2_ref[1]) * d) - h
    acc = jnp.dot(h, w_ref[0], preferred_element_type=jnp.float32)
    acc = acc + jnp.dot(t1, w_ref[1], preferred_element_type=jnp.float32)
    acc = acc + jnp.dot(t2, w_ref[2], preferred_element_type=jnp.float32)
    acc = acc + b_ref[...]
    h1 = _leaky(acc)
    h1_ref[...] = h1
    g2_ref[...] = h1 * d


def _tc_final_body(deg_ref, h_ref, p1_ref, p2_ref, w_ref, b_ref,
                   wp_ref, bp_ref, o_ref):
    d = _dcol(deg_ref)
    h = h_ref[...]
    t1 = -((p1_ref[0] + p1_ref[1]) * d)
    t2 = 2.0 * ((p2_ref[0] + p2_ref[1]) * d) - h
    acc = jnp.dot(h, w_ref[0], preferred_element_type=jnp.float32)
    acc = acc + jnp.dot(t1, w_ref[1], preferred_element_type=jnp.float32)
    acc = acc + jnp.dot(t2, w_ref[2], preferred_element_type=jnp.float32)
    h2 = _leaky(acc + b_ref[...])
    o_ref[...] = (jnp.dot(h2, wp_ref[...], preferred_element_type=jnp.float32)
                  + bp_ref[...])


def _tc_scale(degc, x):
    return pl.pallas_call(
        _tc_scale_body,
        grid=(GRID,),
        in_specs=[_deg_spec, _row_spec],
        out_specs=_row_spec,
        out_shape=jax.ShapeDtypeStruct((N, D), jnp.float32),
    )(degc, x)


def _tc_mid(degc, p):
    return pl.pallas_call(
        _tc_mid_body,
        grid=(GRID,),
        in_specs=[_deg_spec, _par_spec],
        out_specs=_row_spec,
        out_shape=jax.ShapeDtypeStruct((N, D), jnp.float32),
    )(degc, p)


def _tc_combine(degc, h, p1, p2, W, b):
    return pl.pallas_call(
        _tc_combine_body,
        grid=(GRID,),
        in_specs=[_deg_spec, _row_spec, _par_spec, _par_spec,
                  _full((3, D, D)), _full((1, D))],
        out_specs=[_row_spec, _row_spec],
        out_shape=[jax.ShapeDtypeStruct((N, D), jnp.float32),
                   jax.ShapeDtypeStruct((N, D), jnp.float32)],
    )(degc, h, p1, p2, W, b)


def _tc_final(degc, h, p1, p2, W, b, Wp, bp):
    return pl.pallas_call(
        _tc_final_body,
        grid=(GRID,),
        in_specs=[_deg_spec, _row_spec, _par_spec, _par_spec,
                  _full((3, D, D)), _full((1, D)), _full((D, 1)),
                  _full((1, 1))],
        out_specs=pl.BlockSpec((BN, 1), lambda i: (i, 0)),
        out_shape=jax.ShapeDtypeStruct((N, 1), jnp.float32),
    )(degc, h, p1, p2, W, b, Wp, bp)


def kernel(features, edge_index, W0, b0, W1, b1, Wp, bp):
    src = edge_index[0]
    dst = edge_index[1]
    pad = E_PAD - E
    pad_src = (jnp.arange(pad, dtype=jnp.int32) * 131) % N
    pad_dst = N + (jnp.arange(pad, dtype=jnp.int32) % 64)
    src3 = jnp.concatenate([src, pad_src]).reshape(NW, CH, CK)
    dst3 = jnp.concatenate([dst, pad_dst]).reshape(NW, CH, CK)
    zeros1d = jnp.zeros((N_PAD,), jnp.float32)

    degp = _sc_degree()(dst3, zeros1d).reshape(NC, N_PAD)
    degc = degp[:, :N, None]

    spmm = _sc_spmm()
    g0 = _tc_scale(degc, features)
    p1 = spmm(g0, src3, dst3)
    g1 = _tc_mid(degc, p1)
    p2 = spmm(g1, src3, dst3)
    h1, g2 = _tc_combine(degc, features, p1, p2, W0, b0.reshape(1, D))

    q1 = spmm(g2, src3, dst3)
    g3 = _tc_mid(degc, q1)
    q2 = spmm(g3, src3, dst3)
    return _tc_final(degc, h1, q1, q2, W1, b1.reshape(1, D),
                     Wp, bp.reshape(1, 1))
